# Initial kernel scaffold; baseline (speedup 1.0000x reference)
import jax
import jax.numpy as jnp
from jax.experimental import pallas as pl


def kernel(input, pe_weight):
    raise NotImplementedError("write your pallas kernel here")


# speedup vs baseline: 1.7854x; 1.7854x over previous
import jax
import jax.numpy as jnp
from jax.experimental import pallas as pl


_TB = 512


def _body(in_ref, pe_ref, out_ref):
    out_ref[...] = in_ref[...] + jnp.transpose(pe_ref[...], (1, 0))[None]


def kernel(input, pe_weight):
    B, D, T = input.shape
    num_embeds = pe_weight.shape[0]
    tb = _TB
    nwrap = num_embeds // tb
    return pl.pallas_call(
        _body,
        grid=(T // tb, B),
        in_specs=[
            pl.BlockSpec((1, D, tb), lambda t, b: (b, 0, t)),
            pl.BlockSpec((tb, D), lambda t, b: (t % nwrap, 0)),
        ],
        out_specs=pl.BlockSpec((1, D, tb), lambda t, b: (b, 0, t)),
        out_shape=jax.ShapeDtypeStruct(input.shape, input.dtype),
    )(input, pe_weight)


# speedup vs baseline: 2.0157x; 1.1290x over previous
import jax
import jax.numpy as jnp
from jax.experimental import pallas as pl


_TB = 1024


def _body(in_ref, pe_ref, out_ref):
    out_ref[...] = in_ref[...] + jnp.transpose(pe_ref[...], (1, 0))[None]


def kernel(input, pe_weight):
    B, D, T = input.shape
    num_embeds = pe_weight.shape[0]
    tb = _TB
    nwrap = num_embeds // tb
    return pl.pallas_call(
        _body,
        grid=(T // tb, B),
        in_specs=[
            pl.BlockSpec((1, D, tb), lambda t, b: (b, 0, t)),
            pl.BlockSpec((tb, D), lambda t, b: (t % nwrap, 0)),
        ],
        out_specs=pl.BlockSpec((1, D, tb), lambda t, b: (b, 0, t)),
        out_shape=jax.ShapeDtypeStruct(input.shape, input.dtype),
    )(input, pe_weight)


# speedup vs baseline: 2.1320x; 1.0577x over previous
import jax
import jax.numpy as jnp
from jax.experimental import pallas as pl


_TB = 2048


def _body(in_ref, pe_ref, out_ref):
    out_ref[...] = in_ref[...] + jnp.transpose(pe_ref[...], (1, 0))[None]


def kernel(input, pe_weight):
    B, D, T = input.shape
    num_embeds = pe_weight.shape[0]
    tb = _TB
    nwrap = num_embeds // tb
    return pl.pallas_call(
        _body,
        grid=(T // tb, B),
        in_specs=[
            pl.BlockSpec((1, D, tb), lambda t, b: (b, 0, t)),
            pl.BlockSpec((tb, D), lambda t, b: (t % nwrap, 0)),
        ],
        out_specs=pl.BlockSpec((1, D, tb), lambda t, b: (b, 0, t)),
        out_shape=jax.ShapeDtypeStruct(input.shape, input.dtype),
    )(input, pe_weight)
